# Initial kernel scaffold; baseline (speedup 1.0000x reference)
import jax
import jax.numpy as jnp
from jax.experimental import pallas as pl


def kernel(x, token_table, position_table):
    raise NotImplementedError("write your pallas kernel here")


# speedup vs baseline: 2.3646x; 2.3646x over previous
import functools

import jax
import jax.numpy as jnp
from jax import lax
from jax.experimental import pallas as pl
from jax.experimental.pallas import tpu as pltpu
from jax.experimental.pallas import tpu_sc as plsc

_NC = 2
_NS = 16
_NW = _NC * _NS


def kernel(x, token_table, position_table):
    B, S = x.shape
    V, D = token_table.shape
    H = S // 2
    seqs_per_w = B // _NW

    x2 = x.reshape(B * 2, H).astype(jnp.int32)

    mesh = plsc.VectorSubcoreMesh(core_axis_name="c", subcore_axis_name="s")

    @functools.partial(
        pl.kernel,
        out_type=jax.ShapeDtypeStruct((B * S, D), jnp.float32),
        mesh=mesh,
        scratch_types=[
            pltpu.VMEM((2, H), jnp.int32),
            pltpu.VMEM((S, D), jnp.float32),
            pltpu.VMEM((S, D), jnp.float32),
            pltpu.SemaphoreType.DMA,
        ],
        compiler_params=pltpu.CompilerParams(use_tc_tiling_on_sc=False),
    )
    def emb(x_hbm, tok_hbm, pos_hbm, out_hbm, idx_v, rows_v, pos_v, sem):
        wid = lax.axis_index("s") * _NC + lax.axis_index("c")
        pltpu.sync_copy(pos_hbm, pos_v)

        @pl.loop(0, seqs_per_w)
        def _chunk(g):
            seq = wid * seqs_per_w + g
            pltpu.sync_copy(x_hbm.at[pl.ds(seq * 2, 2)], idx_v)
            cp0 = pltpu.async_copy(
                tok_hbm.at[idx_v.at[0]], rows_v.at[pl.ds(0, H)], sem)
            cp1 = pltpu.async_copy(
                tok_hbm.at[idx_v.at[1]], rows_v.at[pl.ds(H, H)], sem)
            cp0.wait()
            cp1.wait()

            @pl.loop(0, S)
            def _add(r):
                for c in range(D // 16):
                    sl = pl.ds(c * 16, 16)
                    rows_v[r, sl] = rows_v[r, sl] + pos_v[r, sl]

            pltpu.sync_copy(rows_v, out_hbm.at[pl.ds(seq * S, S)])

    out = emb(x2, token_table, position_table)
    return out.reshape(B, S, D)
